# baseline (speedup 1.0000x reference)
import functools

import jax
import jax.numpy as jnp
from jax import lax
from jax.experimental import pallas as pl
from jax.experimental.pallas import tpu as pltpu
from jax.experimental.pallas import tpu_sc as plsc

B = 16384
K = 20
HID = 64

NC = 2
NS = 16
NW = NC * NS
BPW = B // NW
CHUNK = 32
NCHUNK = BPW // CHUNK
CROWS = CHUNK * K
NSUB = CHUNK // 16


def _sc_dots_body(wemb_hbm, cemb_hbm, widx_hbm, cidx_hbm, out_hbm,
                  widx_v, cidx_v, wrows_v, crows_v, res_v, sem0, sem1):
    wid = lax.axis_index("s") * NC + lax.axis_index("c")
    lanes = lax.iota(jnp.int32, 16)

    def chunk_body(i, _):
        base = wid * BPW + i * CHUNK
        pltpu.sync_copy(widx_hbm.at[pl.ds(base, CHUNK)], widx_v)
        pltpu.sync_copy(cidx_hbm.at[pl.ds(base * K, CROWS)], cidx_v)
        pltpu.async_copy(wemb_hbm.at[widx_v], wrows_v, sem0).wait()
        for j in range(CROWS // 128):
            pltpu.async_copy(cemb_hbm.at[cidx_v.at[pl.ds(j * 128, 128)]],
                             crows_v.at[pl.ds(j * 128, 128)], sem1).wait()

        for sub in range(NSUB):
            brow = lanes + sub * 16
            ridx = [brow * K + k for k in range(K)]

            def h_body(h, accs):
                hcol = jnp.full((16,), h, jnp.int32)
                wtv = plsc.load_gather(wrows_v, [brow, hcol])
                return tuple(
                    accs[k] + wtv * plsc.load_gather(crows_v, [ridx[k], hcol])
                    for k in range(K))

            accs = lax.fori_loop(
                0, HID, h_body,
                tuple(jnp.zeros((16,), jnp.float32) for _ in range(K)))
            for k in range(K):
                res_v[k, pl.ds(i * CHUNK + sub * 16, 16)] = accs[k]
        return _

    lax.fori_loop(0, NCHUNK, chunk_body, 0)
    for k in range(K):
        pltpu.sync_copy(res_v.at[k], out_hbm.at[pl.ds(k * B + wid * BPW, BPW)])


@jax.jit
def _sc_dots(word_emb, context_emb, widx, cidx):
    mesh = plsc.VectorSubcoreMesh(core_axis_name="c", subcore_axis_name="s",
                                  num_cores=NC, num_subcores=NS)
    return pl.kernel(
        _sc_dots_body,
        out_type=jax.ShapeDtypeStruct((K * B,), jnp.float32),
        mesh=mesh,
        compiler_params=pltpu.CompilerParams(needs_layout_passes=False,
                                             use_tc_tiling_on_sc=False),
        scratch_types=[
            pltpu.VMEM((CHUNK,), jnp.int32),
            pltpu.VMEM((CROWS,), jnp.int32),
            pltpu.VMEM((CHUNK, HID), jnp.float32),
            pltpu.VMEM((CROWS, HID), jnp.float32),
            pltpu.VMEM((K, BPW), jnp.float32),
            pltpu.SemaphoreType.DMA,
            pltpu.SemaphoreType.DMA,
        ],
    )(word_emb, context_emb, widx, cidx)


def _loss_body(res_ref, lab_ref, out_ref):
    x = res_ref[...] * lab_ref[...]
    y = jax.nn.log_sigmoid(x)
    out_ref[0, 0] = -jnp.sum(y) / B


def _loss(res2d, lab2d):
    out = pl.pallas_call(
        _loss_body,
        out_shape=jax.ShapeDtypeStruct((1, 1), jnp.float32),
        in_specs=[pl.BlockSpec(memory_space=pltpu.VMEM),
                  pl.BlockSpec(memory_space=pltpu.VMEM)],
        out_specs=pl.BlockSpec(memory_space=pltpu.SMEM),
    )(res2d, lab2d)
    return out[0, 0]


def kernel(wrd, cntxt, labels, word_emb, context_emb):
    widx = wrd.reshape(B).astype(jnp.int32)
    cidx = cntxt.reshape(B * K).astype(jnp.int32)
    res = _sc_dots(word_emb, context_emb, widx, cidx)
    res2d = res.reshape(K * B // 128, 128)
    lab2d = labels.T.reshape(K * B // 128, 128)
    return _loss(res2d, lab2d)
